# baseline (speedup 1.0000x reference)
import functools

import jax
import jax.numpy as jnp
from jax import lax
from jax.experimental import pallas as pl
from jax.experimental.pallas import tpu as pltpu
from jax.experimental.pallas import tpu_sc as plsc

N = 10000
E = 320000
D = 128

NC = 2
NS = 16
NW = NC * NS
CHUNK = 128
NCHUNK = 80
EPT = NCHUNK * CHUNK
EPAD = NW * EPT
NTAB = 10240
RPT = NTAB // NS
DUMP = NTAB - 8

BN = 2000


@functools.partial(
    pl.kernel,
    out_type=jax.ShapeDtypeStruct((NC * NTAB, D), jnp.float32),
    mesh=plsc.VectorSubcoreMesh(core_axis_name="c", subcore_axis_name="s"),
    scratch_types=[
        pltpu.VMEM((NCHUNK // 2, CHUNK), jnp.int32),
        pltpu.VMEM((NCHUNK // 2, CHUNK), jnp.int32),
        pltpu.VMEM((CHUNK, D), jnp.float32),
        pltpu.VMEM((CHUNK, D), jnp.float32),
        pltpu.VMEM_SHARED((NTAB, D), jnp.float32),
        pltpu.SemaphoreType.DMA,
        pltpu.SemaphoreType.DMA,
    ],
)
def _sc_edge_agg(h_hbm, src_hbm, dst_hbm, out_hbm,
                 src_v, dst_v, rows0_v, rows1_v, agg_sh, sem_g, sem_s):
    c = lax.axis_index("c")
    s = lax.axis_index("s")
    tid = s * NC + c

    zeros16 = jnp.zeros((16,), jnp.float32)

    def zero_row(i, carry):
        for j in range(D // 16):
            rows0_v[i, pl.ds(j * 16, 16)] = zeros16
        return carry

    lax.fori_loop(0, CHUNK, zero_row, 0)
    for k in range(RPT // CHUNK):
        pltpu.sync_copy(rows0_v, agg_sh.at[pl.ds(s * RPT + k * CHUNK, CHUNK)])
    plsc.subcore_barrier()

    bufs = (rows0_v, rows1_v)
    half = NCHUNK // 2
    for hh in range(2):
        pltpu.sync_copy(src_hbm.at[tid, pl.ds(hh * half, half)], src_v)
        pltpu.sync_copy(dst_hbm.at[tid, pl.ds(hh * half, half)], dst_v)
        pltpu.async_copy(h_hbm.at[src_v.at[0]], rows0_v, sem_g).wait()

        def two_steps(it, carry):
            for p in range(2):
                i = it * 2 + p
                gd = pltpu.async_copy(h_hbm.at[src_v.at[i + 1]], bufs[1 - p],
                                      sem_g)
                sd = pltpu.async_copy(bufs[p], agg_sh.at[dst_v.at[i]], sem_s,
                                      add=True)
                sd.wait()
                gd.wait()
            return carry

        lax.fori_loop(0, (half - 2) // 2, two_steps, 0)
        gd = pltpu.async_copy(h_hbm.at[src_v.at[half - 1]], rows1_v, sem_g)
        sd = pltpu.async_copy(rows0_v, agg_sh.at[dst_v.at[half - 2]], sem_s,
                              add=True)
        sd.wait()
        gd.wait()
        pltpu.sync_copy(rows1_v, agg_sh.at[dst_v.at[half - 1]], add=True)
    plsc.subcore_barrier()

    row0 = c * NTAB + s * RPT
    pltpu.sync_copy(agg_sh.at[pl.ds(s * RPT, RPT)], out_hbm.at[pl.ds(row0, RPT)])


def _mlp1_body(x_ref, agg_ref, wa_ref, ba_ref, wb_ref, bb_ref, o_ref):
    h = x_ref[...] + agg_ref[0] + agg_ref[1]
    t = jnp.dot(h, wa_ref[...], preferred_element_type=jnp.float32) + ba_ref[...]
    t = jnp.maximum(t, 0.0)
    u = jnp.dot(t, wb_ref[...], preferred_element_type=jnp.float32) + bb_ref[...]
    o_ref[...] = jnp.maximum(u, 0.0)


def _mlp2_body(x_ref, agg_ref, wa_ref, ba_ref, wb_ref, bb_ref,
               wo_ref, bo_ref, o_ref):
    h = x_ref[...] + agg_ref[0] + agg_ref[1]
    t = jnp.dot(h, wa_ref[...], preferred_element_type=jnp.float32) + ba_ref[...]
    t = jnp.maximum(t, 0.0)
    u = jnp.dot(t, wb_ref[...], preferred_element_type=jnp.float32) + bb_ref[...]
    u = jnp.maximum(u, 0.0)
    o_ref[...] = (jnp.dot(u, wo_ref[...], preferred_element_type=jnp.float32)
                  + bo_ref[...])


_row_spec = pl.BlockSpec((BN, D), lambda i: (i, 0))
_agg_spec = pl.BlockSpec((NC, BN, D), lambda i: (0, i, 0))
_w_spec = pl.BlockSpec((D, D), lambda i: (0, 0))
_b_spec = pl.BlockSpec((1, D), lambda i: (0, 0))

_mlp1 = pl.pallas_call(
    _mlp1_body,
    grid=(N // BN,),
    in_specs=[_row_spec, _agg_spec, _w_spec, _b_spec, _w_spec, _b_spec],
    out_specs=_row_spec,
    out_shape=jax.ShapeDtypeStruct((N, D), jnp.float32),
)

_mlp2 = pl.pallas_call(
    _mlp2_body,
    grid=(N // BN,),
    in_specs=[_row_spec, _agg_spec, _w_spec, _b_spec, _w_spec, _b_spec,
              _w_spec, _b_spec],
    out_specs=_row_spec,
    out_shape=jax.ShapeDtypeStruct((N, D), jnp.float32),
)


def kernel(x, edge_index, W1a, b1a, W1b, b1b, W2a, b2a, W2b, b2b, Wo, bo):
    pad = EPAD - E
    src = jnp.concatenate(
        [edge_index[0], jnp.zeros((pad,), jnp.int32)]).reshape(NW, NCHUNK, CHUNK)
    dst = jnp.concatenate(
        [edge_index[1], jnp.full((pad,), DUMP, jnp.int32)]).reshape(NW, NCHUNK, CHUNK)
    agg1 = _sc_edge_agg(x, src, dst).reshape(NC, NTAB, D)
    h1 = _mlp1(x, agg1, W1a, b1a.reshape(1, D), W1b, b1b.reshape(1, D))
    agg2 = _sc_edge_agg(h1, src, dst).reshape(NC, NTAB, D)
    return _mlp2(h1, agg2, W2a, b2a.reshape(1, D), W2b, b2b.reshape(1, D),
                 Wo, bo.reshape(1, D))


# speedup vs baseline: 1.1488x; 1.1488x over previous
import functools

import jax
import jax.numpy as jnp
from jax import lax
from jax.experimental import pallas as pl
from jax.experimental.pallas import tpu as pltpu
from jax.experimental.pallas import tpu_sc as plsc

N = 10000
E = 320000
D = 128

NC = 2
NS = 16
NW = NC * NS
CHUNK = 128
NCHUNK = 80
EPT = NCHUNK * CHUNK
EPAD = NW * EPT
NTAB = 10240
RPT = NTAB // NS
PADW = EPT - E // NW

BN = 2000


@functools.partial(
    pl.kernel,
    out_type=jax.ShapeDtypeStruct((NC * NTAB, D), jnp.float32),
    mesh=plsc.VectorSubcoreMesh(core_axis_name="c", subcore_axis_name="s"),
    scratch_types=[
        pltpu.VMEM((NCHUNK // 2, CHUNK), jnp.int32),
        pltpu.VMEM((NCHUNK // 2, CHUNK), jnp.int32),
        pltpu.VMEM((CHUNK, D), jnp.float32),
        pltpu.VMEM((CHUNK, D), jnp.float32),
        pltpu.VMEM_SHARED((NTAB, D), jnp.float32),
        pltpu.SemaphoreType.DMA,
        pltpu.SemaphoreType.DMA,
    ],
)
def _sc_edge_agg(h_hbm, src_hbm, dst_hbm, out_hbm,
                 src_v, dst_v, rows0_v, rows1_v, agg_sh, sem_g, sem_s):
    c = lax.axis_index("c")
    s = lax.axis_index("s")
    tid = s * NC + c

    zeros16 = jnp.zeros((16,), jnp.float32)

    def zero_row(i, carry):
        for j in range(D // 16):
            rows0_v[i, pl.ds(j * 16, 16)] = zeros16
        return carry

    lax.fori_loop(0, CHUNK, zero_row, 0)
    for k in range(RPT // CHUNK):
        pltpu.sync_copy(rows0_v, agg_sh.at[pl.ds(s * RPT + k * CHUNK, CHUNK)])
    plsc.subcore_barrier()

    bufs = (rows0_v, rows1_v)
    half = NCHUNK // 2
    for hh in range(2):
        pltpu.sync_copy(src_hbm.at[tid, pl.ds(hh * half, half)], src_v)
        pltpu.sync_copy(dst_hbm.at[tid, pl.ds(hh * half, half)], dst_v)
        pltpu.async_copy(h_hbm.at[src_v.at[0]], rows0_v, sem_g).wait()

        def two_steps(it, carry):
            for p in range(2):
                i = it * 2 + p
                gd = pltpu.async_copy(h_hbm.at[src_v.at[i + 1]], bufs[1 - p],
                                      sem_g)
                sd = pltpu.async_copy(bufs[p], agg_sh.at[dst_v.at[i]], sem_s,
                                      add=True)
                sd.wait()
                gd.wait()
            return carry

        lax.fori_loop(0, (half - 2) // 2, two_steps, 0)
        gd = pltpu.async_copy(h_hbm.at[src_v.at[half - 1]], rows1_v, sem_g)
        sd = pltpu.async_copy(rows0_v, agg_sh.at[dst_v.at[half - 2]], sem_s,
                              add=True)
        sd.wait()
        gd.wait()
        pltpu.sync_copy(rows1_v, agg_sh.at[dst_v.at[half - 1]], add=True)
    plsc.subcore_barrier()

    row0 = c * NTAB + s * RPT
    pltpu.sync_copy(agg_sh.at[pl.ds(s * RPT, RPT)], out_hbm.at[pl.ds(row0, RPT)])


def _mlp1_body(x_ref, agg_ref, wa_ref, ba_ref, wb_ref, bb_ref, o_ref):
    h = x_ref[...] + agg_ref[0] + agg_ref[1]
    t = jnp.dot(h, wa_ref[...], preferred_element_type=jnp.float32) + ba_ref[...]
    t = jnp.maximum(t, 0.0)
    u = jnp.dot(t, wb_ref[...], preferred_element_type=jnp.float32) + bb_ref[...]
    o_ref[...] = jnp.maximum(u, 0.0)


def _mlp2_body(x_ref, agg_ref, wa_ref, ba_ref, wb_ref, bb_ref,
               wo_ref, bo_ref, o_ref):
    h = x_ref[...] + agg_ref[0] + agg_ref[1]
    t = jnp.dot(h, wa_ref[...], preferred_element_type=jnp.float32) + ba_ref[...]
    t = jnp.maximum(t, 0.0)
    u = jnp.dot(t, wb_ref[...], preferred_element_type=jnp.float32) + bb_ref[...]
    u = jnp.maximum(u, 0.0)
    o_ref[...] = (jnp.dot(u, wo_ref[...], preferred_element_type=jnp.float32)
                  + bo_ref[...])


_row_spec = pl.BlockSpec((BN, D), lambda i: (i, 0))
_agg_spec = pl.BlockSpec((NC, BN, D), lambda i: (0, i, 0))
_w_spec = pl.BlockSpec((D, D), lambda i: (0, 0))
_b_spec = pl.BlockSpec((1, D), lambda i: (0, 0))

_mlp1 = pl.pallas_call(
    _mlp1_body,
    grid=(N // BN,),
    in_specs=[_row_spec, _agg_spec, _w_spec, _b_spec, _w_spec, _b_spec],
    out_specs=_row_spec,
    out_shape=jax.ShapeDtypeStruct((N, D), jnp.float32),
)

_mlp2 = pl.pallas_call(
    _mlp2_body,
    grid=(N // BN,),
    in_specs=[_row_spec, _agg_spec, _w_spec, _b_spec, _w_spec, _b_spec,
              _w_spec, _b_spec],
    out_specs=_row_spec,
    out_shape=jax.ShapeDtypeStruct((N, D), jnp.float32),
)


def kernel(x, edge_index, W1a, b1a, W1b, b1b, W2a, b2a, W2b, b2b, Wo, bo):
    pad_dst = jnp.broadcast_to(N + jnp.arange(PADW, dtype=jnp.int32), (NW, PADW))
    src = jnp.concatenate(
        [edge_index[0].reshape(NW, E // NW),
         jnp.zeros((NW, PADW), jnp.int32)], axis=1).reshape(NW, NCHUNK, CHUNK)
    dst = jnp.concatenate(
        [edge_index[1].reshape(NW, E // NW), pad_dst],
        axis=1).reshape(NW, NCHUNK, CHUNK)
    agg1 = _sc_edge_agg(x, src, dst).reshape(NC, NTAB, D)
    h1 = _mlp1(x, agg1, W1a, b1a.reshape(1, D), W1b, b1b.reshape(1, D))
    agg2 = _sc_edge_agg(h1, src, dst).reshape(NC, NTAB, D)
    return _mlp2(h1, agg2, W2a, b2a.reshape(1, D), W2b, b2b.reshape(1, D),
                 Wo, bo.reshape(1, D))


# speedup vs baseline: 2.7668x; 2.4084x over previous
import functools

import jax
import jax.numpy as jnp
from jax import lax
from jax.experimental import pallas as pl
from jax.experimental.pallas import tpu as pltpu
from jax.experimental.pallas import tpu_sc as plsc

N = 10000
E = 320000
D = 128

NC = 2
NS = 16
NW = NC * NS
CHUNK = 128
EPT = E // NW
FULL = EPT // CHUNK
TAIL = EPT - FULL * CHUNK
NTAB = 10240
RPT = NTAB // NS

BN = 2000


@functools.partial(
    pl.kernel,
    out_type=jax.ShapeDtypeStruct((NC * NTAB, D), jnp.float32),
    mesh=plsc.VectorSubcoreMesh(core_axis_name="c", subcore_axis_name="s"),
    scratch_types=[
        pltpu.VMEM((CHUNK,), jnp.int32),
        pltpu.VMEM((CHUNK,), jnp.int32),
        pltpu.VMEM((CHUNK,), jnp.int32),
        pltpu.VMEM((CHUNK,), jnp.int32),
        pltpu.VMEM((TAIL,), jnp.int32),
        pltpu.VMEM((TAIL,), jnp.int32),
        pltpu.VMEM((CHUNK, D), jnp.float32),
        pltpu.VMEM((CHUNK, D), jnp.float32),
        pltpu.VMEM_SHARED((NTAB, D), jnp.float32),
        pltpu.SemaphoreType.DMA,
    ],
)
def _sc_edge_agg(h_hbm, src_hbm, dst_hbm, out_hbm,
                 srcA, srcB, dstA, dstB, srcT, dstT,
                 rows0_v, rows1_v, agg_sh, sem_g):
    c = lax.axis_index("c")
    s = lax.axis_index("s")
    tid = s * NC + c

    zeros16 = jnp.zeros((16,), jnp.float32)

    def zero_row(i, carry):
        for j in range(D // 16):
            rows0_v[i, pl.ds(j * 16, 16)] = zeros16
        return carry

    lax.fori_loop(0, CHUNK, zero_row, 0)
    for k in range(RPT // CHUNK):
        pltpu.sync_copy(rows0_v, agg_sh.at[pl.ds(s * RPT + k * CHUNK, CHUNK)])
    plsc.subcore_barrier()

    base = tid * EPT
    srcs = (srcA, srcB)
    dsts = (dstA, dstB)
    bufs = (rows0_v, rows1_v)

    off0 = pl.multiple_of(base, 8)
    pltpu.sync_copy(src_hbm.at[pl.ds(off0, CHUNK)], srcA)
    pltpu.sync_copy(dst_hbm.at[pl.ds(off0, CHUNK)], dstA)
    pltpu.async_copy(h_hbm.at[srcA], rows0_v, sem_g).wait()

    def two_steps(it, carry):
        for p in range(2):
            i = it * 2 + p
            off = pl.multiple_of(base + (i + 1) * CHUNK, 8)
            pltpu.sync_copy(src_hbm.at[pl.ds(off, CHUNK)], srcs[1 - p])
            gd = pltpu.async_copy(h_hbm.at[srcs[1 - p]], bufs[1 - p], sem_g)
            pltpu.sync_copy(dst_hbm.at[pl.ds(off, CHUNK)], dsts[1 - p])
            pltpu.sync_copy(bufs[p], agg_sh.at[dsts[p]], add=True)
            gd.wait()
        return carry

    lax.fori_loop(0, (FULL - 2) // 2, two_steps, 0)
    offl = pl.multiple_of(base + (FULL - 1) * CHUNK, 8)
    pltpu.sync_copy(src_hbm.at[pl.ds(offl, CHUNK)], srcB)
    gd = pltpu.async_copy(h_hbm.at[srcB], rows1_v, sem_g)
    pltpu.sync_copy(dst_hbm.at[pl.ds(offl, CHUNK)], dstB)
    pltpu.sync_copy(rows0_v, agg_sh.at[dstA], add=True)
    gd.wait()
    offt = pl.multiple_of(base + FULL * CHUNK, 8)
    pltpu.sync_copy(src_hbm.at[pl.ds(offt, TAIL)], srcT)
    gd = pltpu.async_copy(h_hbm.at[srcT], rows0_v.at[pl.ds(0, TAIL)], sem_g)
    pltpu.sync_copy(dst_hbm.at[pl.ds(offt, TAIL)], dstT)
    pltpu.sync_copy(rows1_v, agg_sh.at[dstB], add=True)
    gd.wait()
    pltpu.sync_copy(rows0_v.at[pl.ds(0, TAIL)], agg_sh.at[dstT], add=True)
    plsc.subcore_barrier()

    row0 = c * NTAB + s * RPT
    pltpu.sync_copy(agg_sh.at[pl.ds(s * RPT, RPT)], out_hbm.at[pl.ds(row0, RPT)])


def _mlp1_body(x_ref, agg_ref, wa_ref, ba_ref, wb_ref, bb_ref, o_ref):
    h = x_ref[...] + agg_ref[0] + agg_ref[1]
    t = jnp.dot(h, wa_ref[...], preferred_element_type=jnp.float32) + ba_ref[...]
    t = jnp.maximum(t, 0.0)
    u = jnp.dot(t, wb_ref[...], preferred_element_type=jnp.float32) + bb_ref[...]
    o_ref[...] = jnp.maximum(u, 0.0)


def _mlp2_body(x_ref, agg_ref, wa_ref, ba_ref, wb_ref, bb_ref,
               wo_ref, bo_ref, o_ref):
    h = x_ref[...] + agg_ref[0] + agg_ref[1]
    t = jnp.dot(h, wa_ref[...], preferred_element_type=jnp.float32) + ba_ref[...]
    t = jnp.maximum(t, 0.0)
    u = jnp.dot(t, wb_ref[...], preferred_element_type=jnp.float32) + bb_ref[...]
    u = jnp.maximum(u, 0.0)
    o_ref[...] = (jnp.dot(u, wo_ref[...], preferred_element_type=jnp.float32)
                  + bo_ref[...])


_row_spec = pl.BlockSpec((BN, D), lambda i: (i, 0))
_agg_spec = pl.BlockSpec((NC, BN, D), lambda i: (0, i, 0))
_w_spec = pl.BlockSpec((D, D), lambda i: (0, 0))
_b_spec = pl.BlockSpec((1, D), lambda i: (0, 0))

_mlp1 = pl.pallas_call(
    _mlp1_body,
    grid=(N // BN,),
    in_specs=[_row_spec, _agg_spec, _w_spec, _b_spec, _w_spec, _b_spec],
    out_specs=_row_spec,
    out_shape=jax.ShapeDtypeStruct((N, D), jnp.float32),
)

_mlp2 = pl.pallas_call(
    _mlp2_body,
    grid=(N // BN,),
    in_specs=[_row_spec, _agg_spec, _w_spec, _b_spec, _w_spec, _b_spec,
              _w_spec, _b_spec],
    out_specs=_row_spec,
    out_shape=jax.ShapeDtypeStruct((N, D), jnp.float32),
)


def kernel(x, edge_index, W1a, b1a, W1b, b1b, W2a, b2a, W2b, b2b, Wo, bo):
    src = edge_index[0]
    dst = edge_index[1]
    agg1 = _sc_edge_agg(x, src, dst).reshape(NC, NTAB, D)
    h1 = _mlp1(x, agg1, W1a, b1a.reshape(1, D), W1b, b1b.reshape(1, D))
    agg2 = _sc_edge_agg(h1, src, dst).reshape(NC, NTAB, D)
    return _mlp2(h1, agg2, W2a, b2a.reshape(1, D), W2b, b2b.reshape(1, D),
                 Wo, bo.reshape(1, D))


# speedup vs baseline: 3.3520x; 1.2115x over previous
import functools

import jax
import jax.numpy as jnp
from jax import lax
from jax.experimental import pallas as pl
from jax.experimental.pallas import tpu as pltpu
from jax.experimental.pallas import tpu_sc as plsc

N = 10000
E = 320000
D = 128

NC = 2
NS = 16
NW = NC * NS
CHUNK = 128
EPT = E // NW
FULL = EPT // CHUNK
TAIL = EPT - FULL * CHUNK
NTAB = 10240
RPT = NTAB // NS

BN = 2000


@functools.partial(
    pl.kernel,
    out_type=jax.ShapeDtypeStruct((NC * NTAB, D), jnp.float32),
    mesh=plsc.VectorSubcoreMesh(core_axis_name="c", subcore_axis_name="s"),
    scratch_types=[
        pltpu.VMEM((CHUNK,), jnp.int32),
        pltpu.VMEM((CHUNK,), jnp.int32),
        pltpu.VMEM((CHUNK,), jnp.int32),
        pltpu.VMEM((CHUNK,), jnp.int32),
        pltpu.VMEM((TAIL,), jnp.int32),
        pltpu.VMEM((TAIL,), jnp.int32),
        pltpu.VMEM((CHUNK, D), jnp.float32),
        pltpu.VMEM((CHUNK, D), jnp.float32),
        pltpu.VMEM_SHARED((NTAB, D), jnp.float32),
        pltpu.SemaphoreType.DMA,
        pltpu.SemaphoreType.DMA,
    ],
)
def _sc_edge_agg(h_hbm, src_hbm, dst_hbm, out_hbm,
                 srcA, srcB, dstA, dstB, srcT, dstT,
                 rows0_v, rows1_v, agg_sh, sem_g, sem_i):
    c = lax.axis_index("c")
    s = lax.axis_index("s")
    tid = s * NC + c

    zeros16 = jnp.zeros((16,), jnp.float32)

    def zero_row(i, carry):
        for j in range(D // 16):
            rows0_v[i, pl.ds(j * 16, 16)] = zeros16
        return carry

    lax.fori_loop(0, CHUNK, zero_row, 0)
    for k in range(RPT // CHUNK):
        pltpu.sync_copy(rows0_v, agg_sh.at[pl.ds(s * RPT + k * CHUNK, CHUNK)])
    plsc.subcore_barrier()

    base = tid * EPT
    srcs = (srcA, srcB)
    dsts = (dstA, dstB)
    bufs = (rows0_v, rows1_v)

    def idx_off(i):
        return pl.multiple_of(base + i * CHUNK, 8)

    def idx_issue(i, p):
        off = idx_off(i)
        pltpu.async_copy(src_hbm.at[pl.ds(off, CHUNK)], srcs[p], sem_i)
        pltpu.async_copy(dst_hbm.at[pl.ds(off, CHUNK)], dsts[p], sem_i)

    def idx_wait(i, p):
        off = idx_off(i)
        pltpu.make_async_copy(src_hbm.at[pl.ds(off, CHUNK)], srcs[p], sem_i).wait()
        pltpu.make_async_copy(dst_hbm.at[pl.ds(off, CHUNK)], dsts[p], sem_i).wait()

    pltpu.sync_copy(src_hbm.at[pl.ds(idx_off(0), CHUNK)], srcA)
    pltpu.sync_copy(dst_hbm.at[pl.ds(idx_off(0), CHUNK)], dstA)
    idx_issue(1, 1)
    pltpu.async_copy(h_hbm.at[srcA], rows0_v, sem_g).wait()

    def step(i, p, prefetch):
        idx_wait(i + 1, 1 - p)
        gd = pltpu.async_copy(h_hbm.at[srcs[1 - p]], bufs[1 - p], sem_g)
        pltpu.sync_copy(bufs[p], agg_sh.at[dsts[p]], add=True)
        if prefetch:
            idx_issue(i + 2, p)
        gd.wait()

    def two_steps(it, carry):
        for p in range(2):
            step(it * 2 + p, p, True)
        return carry

    lax.fori_loop(0, (FULL - 4) // 2, two_steps, 0)
    step(FULL - 4, 0, True)
    step(FULL - 3, 1, True)
    step(FULL - 2, 0, False)
    offt = pl.multiple_of(base + FULL * CHUNK, 8)
    pltpu.async_copy(src_hbm.at[pl.ds(offt, TAIL)], srcT, sem_i)
    pltpu.async_copy(dst_hbm.at[pl.ds(offt, TAIL)], dstT, sem_i)
    pltpu.make_async_copy(src_hbm.at[pl.ds(offt, TAIL)], srcT, sem_i).wait()
    gd = pltpu.async_copy(h_hbm.at[srcT], rows0_v.at[pl.ds(0, TAIL)], sem_g)
    pltpu.sync_copy(bufs[1], agg_sh.at[dstB], add=True)
    pltpu.make_async_copy(dst_hbm.at[pl.ds(offt, TAIL)], dstT, sem_i).wait()
    gd.wait()
    pltpu.sync_copy(rows0_v.at[pl.ds(0, TAIL)], agg_sh.at[dstT], add=True)
    plsc.subcore_barrier()

    row0 = c * NTAB + s * RPT
    pltpu.sync_copy(agg_sh.at[pl.ds(s * RPT, RPT)], out_hbm.at[pl.ds(row0, RPT)])


def _mlp1_body(x_ref, agg_ref, wa_ref, ba_ref, wb_ref, bb_ref, o_ref):
    h = x_ref[...] + agg_ref[0] + agg_ref[1]
    t = jnp.dot(h, wa_ref[...], preferred_element_type=jnp.float32) + ba_ref[...]
    t = jnp.maximum(t, 0.0)
    u = jnp.dot(t, wb_ref[...], preferred_element_type=jnp.float32) + bb_ref[...]
    o_ref[...] = jnp.maximum(u, 0.0)


def _mlp2_body(x_ref, agg_ref, wa_ref, ba_ref, wb_ref, bb_ref,
               wo_ref, bo_ref, o_ref):
    h = x_ref[...] + agg_ref[0] + agg_ref[1]
    t = jnp.dot(h, wa_ref[...], preferred_element_type=jnp.float32) + ba_ref[...]
    t = jnp.maximum(t, 0.0)
    u = jnp.dot(t, wb_ref[...], preferred_element_type=jnp.float32) + bb_ref[...]
    u = jnp.maximum(u, 0.0)
    o_ref[...] = (jnp.dot(u, wo_ref[...], preferred_element_type=jnp.float32)
                  + bo_ref[...])


_row_spec = pl.BlockSpec((BN, D), lambda i: (i, 0))
_agg_spec = pl.BlockSpec((NC, BN, D), lambda i: (0, i, 0))
_w_spec = pl.BlockSpec((D, D), lambda i: (0, 0))
_b_spec = pl.BlockSpec((1, D), lambda i: (0, 0))

_mlp1 = pl.pallas_call(
    _mlp1_body,
    grid=(N // BN,),
    in_specs=[_row_spec, _agg_spec, _w_spec, _b_spec, _w_spec, _b_spec],
    out_specs=_row_spec,
    out_shape=jax.ShapeDtypeStruct((N, D), jnp.float32),
)

_mlp2 = pl.pallas_call(
    _mlp2_body,
    grid=(N // BN,),
    in_specs=[_row_spec, _agg_spec, _w_spec, _b_spec, _w_spec, _b_spec,
              _w_spec, _b_spec],
    out_specs=_row_spec,
    out_shape=jax.ShapeDtypeStruct((N, D), jnp.float32),
)


def kernel(x, edge_index, W1a, b1a, W1b, b1b, W2a, b2a, W2b, b2b, Wo, bo):
    src = edge_index[0]
    dst = edge_index[1]
    agg1 = _sc_edge_agg(x, src, dst).reshape(NC, NTAB, D)
    h1 = _mlp1(x, agg1, W1a, b1a.reshape(1, D), W1b, b1b.reshape(1, D))
    agg2 = _sc_edge_agg(h1, src, dst).reshape(NC, NTAB, D)
    return _mlp2(h1, agg2, W2a, b2a.reshape(1, D), W2b, b2b.reshape(1, D),
                 Wo, bo.reshape(1, D))


# speedup vs baseline: 3.3792x; 1.0081x over previous
import functools

import jax
import jax.numpy as jnp
from jax import lax
from jax.experimental import pallas as pl
from jax.experimental.pallas import tpu as pltpu
from jax.experimental.pallas import tpu_sc as plsc

N = 10000
E = 320000
D = 128

NC = 2
NS = 16
NW = NC * NS
CHUNK = 128
EPT = E // NW
FULL = EPT // CHUNK
TAIL = EPT - FULL * CHUNK
NTAB = 10240
RPT = NTAB // NS

BN = 2000


@functools.partial(
    pl.kernel,
    out_type=jax.ShapeDtypeStruct((NC * NTAB, D), jnp.float32),
    mesh=plsc.VectorSubcoreMesh(core_axis_name="c", subcore_axis_name="s"),
    scratch_types=[
        pltpu.VMEM((CHUNK,), jnp.int32),
        pltpu.VMEM((CHUNK,), jnp.int32),
        pltpu.VMEM((CHUNK,), jnp.int32),
        pltpu.VMEM((CHUNK,), jnp.int32),
        pltpu.VMEM((CHUNK,), jnp.int32),
        pltpu.VMEM((CHUNK,), jnp.int32),
        pltpu.VMEM((TAIL,), jnp.int32),
        pltpu.VMEM((TAIL,), jnp.int32),
        pltpu.VMEM((CHUNK, D), jnp.float32),
        pltpu.VMEM((CHUNK, D), jnp.float32),
        pltpu.VMEM_SHARED((NTAB, D), jnp.float32),
        pltpu.SemaphoreType.DMA,
        pltpu.SemaphoreType.DMA,
        pltpu.SemaphoreType.DMA,
    ],
)
def _sc_edge_agg(h_hbm, src_hbm, dst_hbm, out_hbm,
                 srcA, srcB, srcC, dstA, dstB, dstC, srcT, dstT,
                 rows0_v, rows1_v, agg_sh, sem_g, sem_i, sem_s):
    c = lax.axis_index("c")
    s = lax.axis_index("s")
    tid = s * NC + c
    base = tid * EPT
    srcs = (srcA, srcB, srcC)
    dsts = (dstA, dstB, dstC)
    bufs = (rows0_v, rows1_v)

    def idx_off(i):
        return pl.multiple_of(base + i * CHUNK, 8)

    def idx_issue(i, q):
        off = idx_off(i)
        pltpu.async_copy(src_hbm.at[pl.ds(off, CHUNK)], srcs[q], sem_i)
        pltpu.async_copy(dst_hbm.at[pl.ds(off, CHUNK)], dsts[q], sem_i)

    def idx_wait(i, q):
        off = idx_off(i)
        pltpu.make_async_copy(src_hbm.at[pl.ds(off, CHUNK)], srcs[q], sem_i).wait()
        pltpu.make_async_copy(dst_hbm.at[pl.ds(off, CHUNK)], dsts[q], sem_i).wait()

    def scat_issue(rp, q):
        pltpu.async_copy(bufs[rp], agg_sh.at[dsts[q]], sem_s, add=True)

    def scat_wait(rp, q):
        pltpu.make_async_copy(bufs[rp], agg_sh.at[dsts[q]], sem_s).wait()

    idx_issue(0, 0)
    idx_issue(1, 1)
    zeros16 = jnp.zeros((16,), jnp.float32)

    def zero_row(i, carry):
        for j in range(D // 16):
            rows1_v[i, pl.ds(j * 16, 16)] = zeros16
        return carry

    lax.fori_loop(0, CHUNK, zero_row, 0)
    for k in range(RPT // CHUNK):
        pltpu.sync_copy(rows1_v, agg_sh.at[pl.ds(s * RPT + k * CHUNK, CHUNK)])
    idx_wait(0, 0)
    gd0 = pltpu.async_copy(h_hbm.at[srcA], rows0_v, sem_g)
    plsc.subcore_barrier()
    gd0.wait()

    def step(i, k, prefetch=True, first=False):
        rp = k % 2
        idx_wait(i + 1, (k + 1) % 3)
        if not first:
            scat_wait(1 - rp, (k - 1) % 3)
        gd = pltpu.async_copy(h_hbm.at[srcs[(k + 1) % 3]], bufs[1 - rp], sem_g)
        scat_issue(rp, k % 3)
        if prefetch:
            idx_issue(i + 2, (k + 2) % 3)
        gd.wait()

    step(0, 0, first=True)

    def six_steps(it, carry):
        for kk in range(6):
            step(1 + it * 6 + kk, 1 + kk)
        return carry

    lax.fori_loop(0, (FULL - 6) // 6, six_steps, 0)
    step(FULL - 5, FULL - 5)
    step(FULL - 4, FULL - 4)
    step(FULL - 3, FULL - 3)
    step(FULL - 2, FULL - 2, prefetch=False)
    offt = pl.multiple_of(base + FULL * CHUNK, 8)
    pltpu.async_copy(src_hbm.at[pl.ds(offt, TAIL)], srcT, sem_i)
    pltpu.async_copy(dst_hbm.at[pl.ds(offt, TAIL)], dstT, sem_i)
    pltpu.make_async_copy(src_hbm.at[pl.ds(offt, TAIL)], srcT, sem_i).wait()
    scat_wait(0, (FULL - 2) % 3)
    gd = pltpu.async_copy(h_hbm.at[srcT], rows0_v.at[pl.ds(0, TAIL)], sem_g)
    scat_issue(1, (FULL - 1) % 3)
    pltpu.make_async_copy(dst_hbm.at[pl.ds(offt, TAIL)], dstT, sem_i).wait()
    gd.wait()
    scat_wait(1, (FULL - 1) % 3)
    pltpu.sync_copy(rows0_v.at[pl.ds(0, TAIL)], agg_sh.at[dstT], add=True)
    plsc.subcore_barrier()

    row0 = c * NTAB + s * RPT
    pltpu.sync_copy(agg_sh.at[pl.ds(s * RPT, RPT)], out_hbm.at[pl.ds(row0, RPT)])


def _mlp1_body(x_ref, agg_ref, wa_ref, ba_ref, wb_ref, bb_ref, o_ref):
    h = x_ref[...] + agg_ref[0] + agg_ref[1]
    t = jnp.dot(h, wa_ref[...], preferred_element_type=jnp.float32) + ba_ref[...]
    t = jnp.maximum(t, 0.0)
    u = jnp.dot(t, wb_ref[...], preferred_element_type=jnp.float32) + bb_ref[...]
    o_ref[...] = jnp.maximum(u, 0.0)


def _mlp2_body(x_ref, agg_ref, wa_ref, ba_ref, wb_ref, bb_ref,
               wo_ref, bo_ref, o_ref):
    h = x_ref[...] + agg_ref[0] + agg_ref[1]
    t = jnp.dot(h, wa_ref[...], preferred_element_type=jnp.float32) + ba_ref[...]
    t = jnp.maximum(t, 0.0)
    u = jnp.dot(t, wb_ref[...], preferred_element_type=jnp.float32) + bb_ref[...]
    u = jnp.maximum(u, 0.0)
    o_ref[...] = (jnp.dot(u, wo_ref[...], preferred_element_type=jnp.float32)
                  + bo_ref[...])


_row_spec = pl.BlockSpec((BN, D), lambda i: (i, 0))
_agg_spec = pl.BlockSpec((NC, BN, D), lambda i: (0, i, 0))
_w_spec = pl.BlockSpec((D, D), lambda i: (0, 0))
_b_spec = pl.BlockSpec((1, D), lambda i: (0, 0))

_mlp1 = pl.pallas_call(
    _mlp1_body,
    grid=(N // BN,),
    in_specs=[_row_spec, _agg_spec, _w_spec, _b_spec, _w_spec, _b_spec],
    out_specs=_row_spec,
    out_shape=jax.ShapeDtypeStruct((N, D), jnp.float32),
)

_mlp2 = pl.pallas_call(
    _mlp2_body,
    grid=(N // BN,),
    in_specs=[_row_spec, _agg_spec, _w_spec, _b_spec, _w_spec, _b_spec,
              _w_spec, _b_spec],
    out_specs=_row_spec,
    out_shape=jax.ShapeDtypeStruct((N, D), jnp.float32),
)


def kernel(x, edge_index, W1a, b1a, W1b, b1b, W2a, b2a, W2b, b2b, Wo, bo):
    src = edge_index[0]
    dst = edge_index[1]
    agg1 = _sc_edge_agg(x, src, dst).reshape(NC, NTAB, D)
    h1 = _mlp1(x, agg1, W1a, b1a.reshape(1, D), W1b, b1b.reshape(1, D))
    agg2 = _sc_edge_agg(h1, src, dst).reshape(NC, NTAB, D)
    return _mlp2(h1, agg2, W2a, b2a.reshape(1, D), W2b, b2b.reshape(1, D),
                 Wo, bo.reshape(1, D))


# speedup vs baseline: 4.4364x; 1.3128x over previous
import functools

import jax
import jax.numpy as jnp
from jax import lax
from jax.experimental import pallas as pl
from jax.experimental.pallas import tpu as pltpu
from jax.experimental.pallas import tpu_sc as plsc

N = 10000
E = 320000
D = 128

NC = 2
NS = 16
NW = NC * NS
CHUNK = 128
EPT = E // NW
FULL = EPT // CHUNK
TAIL = EPT - FULL * CHUNK
NTAB = 10112
RPT = NTAB // NS

BN = 2000


@functools.partial(
    pl.kernel,
    out_type=jax.ShapeDtypeStruct((NC * NTAB, D), jnp.float32),
    mesh=plsc.VectorSubcoreMesh(core_axis_name="c", subcore_axis_name="s"),
    scratch_types=[
        pltpu.VMEM((CHUNK,), jnp.int32),
        pltpu.VMEM((CHUNK,), jnp.int32),
        pltpu.VMEM((CHUNK,), jnp.int32),
        pltpu.VMEM((CHUNK,), jnp.int32),
        pltpu.VMEM((CHUNK,), jnp.int32),
        pltpu.VMEM((CHUNK,), jnp.int32),
        pltpu.VMEM((TAIL,), jnp.int32),
        pltpu.VMEM((TAIL,), jnp.int32),
        pltpu.VMEM((CHUNK, D), jnp.float32),
        pltpu.VMEM((CHUNK, D), jnp.float32),
        pltpu.VMEM((CHUNK, D), jnp.float32),
        pltpu.VMEM_SHARED((NTAB, D), jnp.float32),
        pltpu.SemaphoreType.DMA,
        pltpu.SemaphoreType.DMA,
        pltpu.SemaphoreType.DMA,
        pltpu.SemaphoreType.DMA,
        pltpu.SemaphoreType.DMA,
        pltpu.SemaphoreType.DMA,
        pltpu.SemaphoreType.DMA,
        pltpu.SemaphoreType.DMA,
        pltpu.SemaphoreType.DMA,
        pltpu.SemaphoreType.DMA,
    ],
)
def _sc_edge_agg(h_hbm, src_hbm, dst_hbm, out_hbm,
                 srcA, srcB, srcC, dstA, dstB, dstC, srcT, dstT,
                 rows0_v, rows1_v, rows2_v, agg_sh,
                 sg0, sg1, sg2, si0, si1, si2, sd0, sd1, sd2, sem_s):
    c = lax.axis_index("c")
    s = lax.axis_index("s")
    tid = s * NC + c
    base = tid * EPT
    srcs = (srcA, srcB, srcC)
    dsts = (dstA, dstB, dstC)
    rows = (rows0_v, rows1_v, rows2_v)
    sem_g = (sg0, sg1, sg2)
    sem_si = (si0, si1, si2)
    sem_di = (sd0, sd1, sd2)

    def idx_off(i):
        return pl.multiple_of(base + i * CHUNK, 8)

    def src_issue(i, q):
        pltpu.async_copy(src_hbm.at[pl.ds(idx_off(i), CHUNK)], srcs[q], sem_si[q])

    def src_wait(i, q):
        pltpu.make_async_copy(
            src_hbm.at[pl.ds(idx_off(i), CHUNK)], srcs[q], sem_si[q]).wait()

    def dst_issue(i, q):
        pltpu.async_copy(dst_hbm.at[pl.ds(idx_off(i), CHUNK)], dsts[q], sem_di[q])

    def dst_wait(i, q):
        pltpu.make_async_copy(
            dst_hbm.at[pl.ds(idx_off(i), CHUNK)], dsts[q], sem_di[q]).wait()

    def gath_issue(q):
        return pltpu.async_copy(h_hbm.at[srcs[q]], rows[q], sem_g[q])

    def gath_wait(q):
        pltpu.make_async_copy(h_hbm.at[srcs[q]], rows[q], sem_g[q]).wait()

    def scat_issue(q):
        pltpu.async_copy(rows[q], agg_sh.at[dsts[q]], sem_s, add=True)

    def scat_wait(q):
        pltpu.make_async_copy(rows[q], agg_sh.at[dsts[q]], sem_s).wait()

    src_issue(0, 0)
    src_issue(1, 1)
    src_issue(2, 2)
    dst_issue(0, 0)
    dst_issue(1, 1)
    zeros16 = jnp.zeros((16,), jnp.float32)

    def zero_row(i, carry):
        for j in range(D // 16):
            rows2_v[i, pl.ds(j * 16, 16)] = zeros16
        return carry

    lax.fori_loop(0, CHUNK, zero_row, 0)
    for k in range(RPT // CHUNK):
        pltpu.sync_copy(rows2_v, agg_sh.at[pl.ds(s * RPT + k * CHUNK, CHUNK)])
    pltpu.sync_copy(rows2_v.at[pl.ds(0, RPT % CHUNK)],
                    agg_sh.at[pl.ds(s * RPT + (RPT // CHUNK) * CHUNK,
                                    RPT % CHUNK)])
    src_wait(0, 0)
    gd0 = gath_issue(0)
    src_wait(1, 1)
    gd1 = gath_issue(1)
    plsc.subcore_barrier()
    gd0.wait()
    src_issue(3, 0)

    def step(i, k, first=False, swait=True, siss=True, giss=True,
             gwait=True, diss=True):
        if swait:
            src_wait(i + 2, (k + 2) % 3)
        if not first:
            scat_wait((k - 1) % 3)
        if giss:
            gath_issue((k + 2) % 3)
        dst_wait(i, k % 3)
        scat_issue(k % 3)
        if diss:
            dst_issue(i + 2, (k + 2) % 3)
        if gwait:
            gath_wait((k + 1) % 3)
        if siss:
            src_issue(i + 4, (k + 1) % 3)

    step(0, 0, first=True)

    def three_steps(it, carry):
        for kk in range(3):
            step(1 + it * 3 + kk, 1 + kk)
        return carry

    lax.fori_loop(0, (FULL - 6) // 3, three_steps, 0)
    step(FULL - 5, FULL - 5)
    offt = pl.multiple_of(base + FULL * CHUNK, 8)
    pltpu.async_copy(src_hbm.at[pl.ds(offt, TAIL)], srcT, si0)
    pltpu.async_copy(dst_hbm.at[pl.ds(offt, TAIL)], dstT, sd0)
    step(FULL - 4, FULL - 4, siss=False)
    step(FULL - 3, FULL - 3, siss=False)
    step(FULL - 2, FULL - 2, swait=False, siss=False, giss=False, diss=False)
    step(FULL - 1, FULL - 1, swait=False, siss=False, giss=False,
         gwait=False, diss=False)
    pltpu.make_async_copy(src_hbm.at[pl.ds(offt, TAIL)], srcT, si0).wait()
    gd = pltpu.async_copy(h_hbm.at[srcT], rows0_v.at[pl.ds(0, TAIL)], sg0)
    pltpu.make_async_copy(dst_hbm.at[pl.ds(offt, TAIL)], dstT, sd0).wait()
    gd.wait()
    scat_wait((FULL - 1) % 3)
    pltpu.sync_copy(rows0_v.at[pl.ds(0, TAIL)], agg_sh.at[dstT], add=True)
    plsc.subcore_barrier()

    row0 = c * NTAB + s * RPT
    pltpu.sync_copy(agg_sh.at[pl.ds(s * RPT, RPT)], out_hbm.at[pl.ds(row0, RPT)])


def _mlp1_body(x_ref, agg_ref, wa_ref, ba_ref, wb_ref, bb_ref, o_ref):
    h = x_ref[...] + agg_ref[0] + agg_ref[1]
    t = jnp.dot(h, wa_ref[...], preferred_element_type=jnp.float32) + ba_ref[...]
    t = jnp.maximum(t, 0.0)
    u = jnp.dot(t, wb_ref[...], preferred_element_type=jnp.float32) + bb_ref[...]
    o_ref[...] = jnp.maximum(u, 0.0)


def _mlp2_body(x_ref, agg_ref, wa_ref, ba_ref, wb_ref, bb_ref,
               wo_ref, bo_ref, o_ref):
    h = x_ref[...] + agg_ref[0] + agg_ref[1]
    t = jnp.dot(h, wa_ref[...], preferred_element_type=jnp.float32) + ba_ref[...]
    t = jnp.maximum(t, 0.0)
    u = jnp.dot(t, wb_ref[...], preferred_element_type=jnp.float32) + bb_ref[...]
    u = jnp.maximum(u, 0.0)
    o_ref[...] = (jnp.dot(u, wo_ref[...], preferred_element_type=jnp.float32)
                  + bo_ref[...])


_row_spec = pl.BlockSpec((BN, D), lambda i: (i, 0))
_agg_spec = pl.BlockSpec((NC, BN, D), lambda i: (0, i, 0))
_w_spec = pl.BlockSpec((D, D), lambda i: (0, 0))
_b_spec = pl.BlockSpec((1, D), lambda i: (0, 0))

_mlp1 = pl.pallas_call(
    _mlp1_body,
    grid=(N // BN,),
    in_specs=[_row_spec, _agg_spec, _w_spec, _b_spec, _w_spec, _b_spec],
    out_specs=_row_spec,
    out_shape=jax.ShapeDtypeStruct((N, D), jnp.float32),
)

_mlp2 = pl.pallas_call(
    _mlp2_body,
    grid=(N // BN,),
    in_specs=[_row_spec, _agg_spec, _w_spec, _b_spec, _w_spec, _b_spec,
              _w_spec, _b_spec],
    out_specs=_row_spec,
    out_shape=jax.ShapeDtypeStruct((N, D), jnp.float32),
)


def kernel(x, edge_index, W1a, b1a, W1b, b1b, W2a, b2a, W2b, b2b, Wo, bo):
    src = edge_index[0]
    dst = edge_index[1]
    agg1 = _sc_edge_agg(x, src, dst).reshape(NC, NTAB, D)
    h1 = _mlp1(x, agg1, W1a, b1a.reshape(1, D), W1b, b1b.reshape(1, D))
    agg2 = _sc_edge_agg(h1, src, dst).reshape(NC, NTAB, D)
    return _mlp2(h1, agg2, W2a, b2a.reshape(1, D), W2b, b2b.reshape(1, D),
                 Wo, bo.reshape(1, D))


# speedup vs baseline: 4.5295x; 1.0210x over previous
import functools

import jax
import jax.numpy as jnp
from jax import lax
from jax.experimental import pallas as pl
from jax.experimental.pallas import tpu as pltpu
from jax.experimental.pallas import tpu_sc as plsc

N = 10000
E = 320000
D = 128

NC = 2
NS = 16
NW = NC * NS
CHUNK = 128
EPT = E // NW
FULL = EPT // CHUNK
TAIL = EPT - FULL * CHUNK
NTAB = 10112
RPT = NTAB // NS

BN = 2000


@functools.partial(
    pl.kernel,
    out_type=jax.ShapeDtypeStruct((NC * NTAB, D), jnp.float32),
    mesh=plsc.VectorSubcoreMesh(core_axis_name="c", subcore_axis_name="s"),
    scratch_types=[
        pltpu.VMEM((CHUNK,), jnp.int32),
        pltpu.VMEM((CHUNK,), jnp.int32),
        pltpu.VMEM((CHUNK,), jnp.int32),
        pltpu.VMEM((CHUNK,), jnp.int32),
        pltpu.VMEM((CHUNK,), jnp.int32),
        pltpu.VMEM((CHUNK,), jnp.int32),
        pltpu.VMEM((TAIL,), jnp.int32),
        pltpu.VMEM((TAIL,), jnp.int32),
        pltpu.VMEM((CHUNK, D), jnp.float32),
        pltpu.VMEM((CHUNK, D), jnp.float32),
        pltpu.VMEM((CHUNK, D), jnp.float32),
        pltpu.VMEM_SHARED((NTAB, D), jnp.float32),
        pltpu.SemaphoreType.DMA,
        pltpu.SemaphoreType.DMA,
        pltpu.SemaphoreType.DMA,
        pltpu.SemaphoreType.DMA,
        pltpu.SemaphoreType.DMA,
        pltpu.SemaphoreType.DMA,
        pltpu.SemaphoreType.DMA,
        pltpu.SemaphoreType.DMA,
        pltpu.SemaphoreType.DMA,
        pltpu.SemaphoreType.DMA,
    ],
)
def _sc_edge_agg(h_hbm, src_hbm, dst_hbm, out_hbm,
                 srcA, srcB, srcC, dstA, dstB, dstC, srcT, dstT,
                 rows0_v, rows1_v, rows2_v, agg_sh,
                 sg0, sg1, sg2, si0, si1, si2, sd0, sd1, sd2, sem_s):
    c = lax.axis_index("c")
    s = lax.axis_index("s")
    tid = s * NC + c
    base = tid * EPT
    srcs = (srcA, srcB, srcC)
    dsts = (dstA, dstB, dstC)
    rows = (rows0_v, rows1_v, rows2_v)
    sem_g = (sg0, sg1, sg2)
    sem_si = (si0, si1, si2)
    sem_di = (sd0, sd1, sd2)

    def idx_off(i):
        return pl.multiple_of(base + i * CHUNK, 8)

    def src_issue(i, q):
        pltpu.async_copy(src_hbm.at[pl.ds(idx_off(i), CHUNK)], srcs[q], sem_si[q])

    def src_wait(i, q):
        pltpu.make_async_copy(
            src_hbm.at[pl.ds(idx_off(i), CHUNK)], srcs[q], sem_si[q]).wait()

    def dst_issue(i, q):
        pltpu.async_copy(dst_hbm.at[pl.ds(idx_off(i), CHUNK)], dsts[q], sem_di[q])

    def dst_wait(i, q):
        pltpu.make_async_copy(
            dst_hbm.at[pl.ds(idx_off(i), CHUNK)], dsts[q], sem_di[q]).wait()

    def gath_issue(q):
        return pltpu.async_copy(h_hbm.at[srcs[q]], rows[q], sem_g[q])

    def gath_wait(q):
        pltpu.make_async_copy(h_hbm.at[srcs[q]], rows[q], sem_g[q]).wait()

    def scat_issue(q):
        pltpu.async_copy(rows[q], agg_sh.at[dsts[q]], sem_s, add=True)

    def scat_wait(q):
        pltpu.make_async_copy(rows[q], agg_sh.at[dsts[q]], sem_s).wait()

    src_issue(0, 0)
    src_issue(1, 1)
    src_issue(2, 2)
    dst_issue(0, 0)
    dst_issue(1, 1)
    zeros16 = jnp.zeros((16,), jnp.float32)

    def zero_row(i, carry):
        for j in range(D // 16):
            rows2_v[i, pl.ds(j * 16, 16)] = zeros16
        return carry

    lax.fori_loop(0, CHUNK, zero_row, 0)
    zcop = []
    for k in range(RPT // CHUNK):
        zcop.append(pltpu.async_copy(
            rows2_v, agg_sh.at[pl.ds(s * RPT + k * CHUNK, CHUNK)], sem_s))
    zcop.append(pltpu.async_copy(
        rows2_v.at[pl.ds(0, RPT % CHUNK)],
        agg_sh.at[pl.ds(s * RPT + (RPT // CHUNK) * CHUNK, RPT % CHUNK)],
        sem_s))
    src_wait(0, 0)
    gd0 = gath_issue(0)
    src_wait(1, 1)
    gd1 = gath_issue(1)
    for zd in zcop:
        zd.wait()
    plsc.subcore_barrier()
    gd0.wait()
    src_issue(3, 0)

    def step(i, k, first=False, swait=True, siss=True, giss=True,
             gwait=True, diss=True):
        if swait:
            src_wait(i + 2, (k + 2) % 3)
        if not first:
            scat_wait((k - 1) % 3)
        if giss:
            gath_issue((k + 2) % 3)
        dst_wait(i, k % 3)
        scat_issue(k % 3)
        if diss:
            dst_issue(i + 2, (k + 2) % 3)
        if gwait:
            gath_wait((k + 1) % 3)
        if siss:
            src_issue(i + 4, (k + 1) % 3)

    step(0, 0, first=True)

    def three_steps(it, carry):
        for kk in range(3):
            step(1 + it * 3 + kk, 1 + kk)
        return carry

    lax.fori_loop(0, (FULL - 6) // 3, three_steps, 0)
    step(FULL - 5, FULL - 5)
    offt = pl.multiple_of(base + FULL * CHUNK, 8)
    pltpu.async_copy(src_hbm.at[pl.ds(offt, TAIL)], srcT, si0)
    pltpu.async_copy(dst_hbm.at[pl.ds(offt, TAIL)], dstT, sd0)
    step(FULL - 4, FULL - 4, siss=False)
    step(FULL - 3, FULL - 3, siss=False)
    step(FULL - 2, FULL - 2, swait=False, siss=False, giss=False, diss=False)
    step(FULL - 1, FULL - 1, swait=False, siss=False, giss=False,
         gwait=False, diss=False)
    pltpu.make_async_copy(src_hbm.at[pl.ds(offt, TAIL)], srcT, si0).wait()
    gd = pltpu.async_copy(h_hbm.at[srcT], rows0_v.at[pl.ds(0, TAIL)], sg0)
    pltpu.make_async_copy(dst_hbm.at[pl.ds(offt, TAIL)], dstT, sd0).wait()
    gd.wait()
    scat_wait((FULL - 1) % 3)
    pltpu.sync_copy(rows0_v.at[pl.ds(0, TAIL)], agg_sh.at[dstT], add=True)
    plsc.subcore_barrier()

    row0 = c * NTAB + s * RPT
    pltpu.sync_copy(agg_sh.at[pl.ds(s * RPT, RPT)], out_hbm.at[pl.ds(row0, RPT)])


def _mlp1_body(x_ref, agg_ref, wa_ref, ba_ref, wb_ref, bb_ref, o_ref):
    h = x_ref[...] + agg_ref[0] + agg_ref[1]
    t = jnp.dot(h, wa_ref[...], preferred_element_type=jnp.float32) + ba_ref[...]
    t = jnp.maximum(t, 0.0)
    u = jnp.dot(t, wb_ref[...], preferred_element_type=jnp.float32) + bb_ref[...]
    o_ref[...] = jnp.maximum(u, 0.0)


def _mlp2_body(x_ref, agg_ref, wa_ref, ba_ref, wb_ref, bb_ref,
               wo_ref, bo_ref, o_ref):
    h = x_ref[...] + agg_ref[0] + agg_ref[1]
    t = jnp.dot(h, wa_ref[...], preferred_element_type=jnp.float32) + ba_ref[...]
    t = jnp.maximum(t, 0.0)
    u = jnp.dot(t, wb_ref[...], preferred_element_type=jnp.float32) + bb_ref[...]
    u = jnp.maximum(u, 0.0)
    o_ref[...] = (jnp.dot(u, wo_ref[...], preferred_element_type=jnp.float32)
                  + bo_ref[...])


_row_spec = pl.BlockSpec((BN, D), lambda i: (i, 0))
_agg_spec = pl.BlockSpec((NC, BN, D), lambda i: (0, i, 0))
_w_spec = pl.BlockSpec((D, D), lambda i: (0, 0))
_b_spec = pl.BlockSpec((1, D), lambda i: (0, 0))

_mlp1 = pl.pallas_call(
    _mlp1_body,
    grid=(N // BN,),
    in_specs=[_row_spec, _agg_spec, _w_spec, _b_spec, _w_spec, _b_spec],
    out_specs=_row_spec,
    out_shape=jax.ShapeDtypeStruct((N, D), jnp.float32),
)

_mlp2 = pl.pallas_call(
    _mlp2_body,
    grid=(N // BN,),
    in_specs=[_row_spec, _agg_spec, _w_spec, _b_spec, _w_spec, _b_spec,
              _w_spec, _b_spec],
    out_specs=_row_spec,
    out_shape=jax.ShapeDtypeStruct((N, D), jnp.float32),
)


def kernel(x, edge_index, W1a, b1a, W1b, b1b, W2a, b2a, W2b, b2b, Wo, bo):
    src = edge_index[0]
    dst = edge_index[1]
    agg1 = _sc_edge_agg(x, src, dst).reshape(NC, NTAB, D)
    h1 = _mlp1(x, agg1, W1a, b1a.reshape(1, D), W1b, b1b.reshape(1, D))
    agg2 = _sc_edge_agg(h1, src, dst).reshape(NC, NTAB, D)
    return _mlp2(h1, agg2, W2a, b2a.reshape(1, D), W2b, b2b.reshape(1, D),
                 Wo, bo.reshape(1, D))
